# Initial kernel scaffold; baseline (speedup 1.0000x reference)
import jax
import jax.numpy as jnp
from jax.experimental import pallas as pl


def kernel(boxes, scores):
    raise NotImplementedError("write your pallas kernel here")


# speedup vs baseline: 627.9938x; 627.9938x over previous
import jax
import jax.numpy as jnp
import numpy as np
from jax.experimental import pallas as pl
from jax.experimental.pallas import tpu as pltpu

_IOU_T = 0.5
_SCORE_T = 0.5
_MAX_OUT = 200
_N = 5000
_NPAD = 5120
_BC = 16
_KPAD = 256
_P = _BC * _MAX_OUT


def _nms_kernel(x1_ref, y1_ref, x2_ref, y2_ref, sc_ref, out_ref):
    x1 = x1_ref[...]
    y1 = y1_ref[...]
    x2 = x2_ref[...]
    y2 = y2_ref[...]
    sc = sc_ref[...]
    areas = (x2 - x1) * (y2 - y1)

    iota_n = jax.lax.broadcasted_iota(jnp.int32, (_BC, _NPAD), 1)
    col_iota = jax.lax.broadcasted_iota(jnp.int32, (_BC, _KPAD), 1)

    scm0 = jnp.where(sc > _SCORE_T, sc, -1.0)
    keep0 = jnp.zeros((_BC, _KPAD), jnp.float32)
    cnt0 = jnp.zeros((_BC, 1), jnp.int32)

    def body(k, state):
        scm, keep, cnt = state
        m = jnp.max(scm, axis=1, keepdims=True)
        found = m > _SCORE_T
        ism = (scm == m) & found
        idx = jnp.min(jnp.where(ism, iota_n, _NPAD), axis=1, keepdims=True)
        onehot = iota_n == idx
        bx1 = jnp.sum(jnp.where(onehot, x1, 0.0), axis=1, keepdims=True)
        by1 = jnp.sum(jnp.where(onehot, y1, 0.0), axis=1, keepdims=True)
        bx2 = jnp.sum(jnp.where(onehot, x2, 0.0), axis=1, keepdims=True)
        by2 = jnp.sum(jnp.where(onehot, y2, 0.0), axis=1, keepdims=True)
        barea = jnp.sum(jnp.where(onehot, areas, 0.0), axis=1, keepdims=True)
        w = jnp.maximum(jnp.minimum(x2, bx2) - jnp.maximum(x1, bx1), 0.0)
        h = jnp.maximum(jnp.minimum(y2, by2) - jnp.maximum(y1, by1), 0.0)
        inter = w * h
        union = areas + barea - inter
        sup = inter / union > _IOU_T
        scm = jnp.where((sup | onehot) & found, -1.0, scm)
        keep = keep + jnp.where((col_iota == k) & found,
                                idx.astype(jnp.float32), 0.0)
        cnt = cnt + found.astype(jnp.int32)
        return scm, keep, cnt

    _, keep, cnt = jax.lax.fori_loop(0, _MAX_OUT, body, (scm0, keep0, cnt0),
                                     unroll=2)

    ri = jax.lax.broadcasted_iota(jnp.int32, (_BC, _BC), 0)
    ci = jax.lax.broadcasted_iota(jnp.int32, (_BC, _BC), 1)
    tril = (ci <= ri).astype(jnp.float32)
    cnt_f = cnt.astype(jnp.float32)
    cum_end = jax.lax.dot_general(tril, cnt_f, (((1,), (0,)), ((), ())),
                                  preferred_element_type=jnp.float32)
    cum_end_i = cum_end.astype(jnp.int32)
    cum_start_i = cum_end_i - cnt
    total = jnp.max(cum_end_i)

    p = jax.lax.broadcasted_iota(jnp.int32, (1, _P), 1)
    bc = jnp.sum((p >= cum_end_i).astype(jnp.int32), axis=0, keepdims=True)
    class_iota = jax.lax.broadcasted_iota(jnp.int32, (_BC, _P), 0)
    rowhot = class_iota == bc
    start_p = jnp.sum(jnp.where(rowhot, cum_start_i, 0), axis=0, keepdims=True)
    slot = p - start_p

    keep_hi = jnp.floor(keep * (1.0 / 256.0))
    keep_lo = keep - 256.0 * keep_hi
    rowhot_f = rowhot.astype(jnp.float32)
    dn = (((0,), (0,)), ((), ()))
    cmat_hi = jax.lax.dot_general(keep_hi, rowhot_f, dn,
                                  preferred_element_type=jnp.float32)
    cmat_lo = jax.lax.dot_general(keep_lo, rowhot_f, dn,
                                  preferred_element_type=jnp.float32)
    slot_iota = jax.lax.broadcasted_iota(jnp.int32, (_KPAD, _P), 0)
    slothot = slot_iota == slot
    sel_hi = jnp.sum(jnp.where(slothot, cmat_hi, 0.0), axis=0, keepdims=True)
    sel_lo = jnp.sum(jnp.where(slothot, cmat_lo, 0.0), axis=0, keepdims=True)
    sel = sel_hi.astype(jnp.int32) * 256 + sel_lo.astype(jnp.int32)

    valid = p < total
    b_row = jnp.where(valid, bc >> 3, 0)
    c_row = jnp.where(valid, bc & 7, 0)
    s_row = jnp.where(valid, sel, 0)
    sub = jax.lax.broadcasted_iota(jnp.int32, (8, _P), 0)
    out_ref[...] = jnp.where(sub == 0, b_row,
                             jnp.where(sub == 1, c_row,
                                       jnp.where(sub == 2, s_row, 0)))


def kernel(boxes, scores):
    B, C, N = scores.shape
    sc = scores.reshape(B * C, N)
    sc = jnp.pad(sc, ((0, 0), (0, _NPAD - N)))
    bx = jnp.pad(boxes, ((0, 0), (0, _NPAD - N), (0, 0)))
    x1 = jnp.repeat(bx[:, :, 1], C, axis=0)
    y1 = jnp.repeat(bx[:, :, 0], C, axis=0)
    x2 = jnp.repeat(bx[:, :, 3], C, axis=0)
    y2 = jnp.repeat(bx[:, :, 2], C, axis=0)

    out = pl.pallas_call(
        _nms_kernel,
        out_shape=jax.ShapeDtypeStruct((8, _P), jnp.int32),
    )(x1, y1, x2, y2, sc)

    out_dtype = jnp.asarray(np.zeros((1,), dtype=np.int64)).dtype
    return out[:3].T.astype(out_dtype)


# speedup vs baseline: 699.0204x; 1.1131x over previous
import jax
import jax.numpy as jnp
import numpy as np
from jax.experimental import pallas as pl
from jax.experimental.pallas import tpu as pltpu

_IOU_T = 0.5
_SCORE_T = 0.5
_MAX_OUT = 200
_N = 5000
_NPAD = 5120
_BC = 16
_KPAD = 256
_P = _BC * _MAX_OUT


def _nms_kernel(x1_ref, y1_ref, x2_ref, y2_ref, sc_ref, out_ref):
    x1 = x1_ref[...]
    y1 = y1_ref[...]
    x2 = x2_ref[...]
    y2 = y2_ref[...]
    sc = sc_ref[...]
    areas = (x2 - x1) * (y2 - y1)

    iota_n = jax.lax.broadcasted_iota(jnp.int32, (_BC, _NPAD), 1)
    col_iota = jax.lax.broadcasted_iota(jnp.int32, (_BC, _KPAD), 1)

    scm0 = jnp.where(sc > _SCORE_T, sc, -1.0)
    keep0 = jnp.zeros((_BC, _KPAD), jnp.float32)
    cnt0 = jnp.zeros((_BC, 1), jnp.int32)

    def body(k, state):
        scm, keep, cnt = state
        m = jnp.max(scm, axis=1, keepdims=True)
        found = m > _SCORE_T
        idx = jnp.min(jnp.where(scm == m, iota_n, _NPAD), axis=1, keepdims=True)
        onehot = iota_n == idx
        bx1 = jnp.sum(jnp.where(onehot, x1, 0.0), axis=1, keepdims=True)
        by1 = jnp.sum(jnp.where(onehot, y1, 0.0), axis=1, keepdims=True)
        bx2 = jnp.sum(jnp.where(onehot, x2, 0.0), axis=1, keepdims=True)
        by2 = jnp.sum(jnp.where(onehot, y2, 0.0), axis=1, keepdims=True)
        barea = (bx2 - bx1) * (by2 - by1)
        w = jnp.maximum(jnp.minimum(x2, bx2) - jnp.maximum(x1, bx1), 0.0)
        h = jnp.maximum(jnp.minimum(y2, by2) - jnp.maximum(y1, by1), 0.0)
        inter = w * h
        union = areas + barea - inter
        sup = inter / union > _IOU_T
        scm = jnp.where((sup | onehot) & found, -1.0, scm)
        keep = keep + jnp.where((col_iota == k) & found,
                                idx.astype(jnp.float32), 0.0)
        cnt = cnt + found.astype(jnp.int32)
        return scm, keep, cnt

    _, keep, cnt = jax.lax.fori_loop(0, _MAX_OUT, body, (scm0, keep0, cnt0),
                                     unroll=4)

    ri = jax.lax.broadcasted_iota(jnp.int32, (_BC, _BC), 0)
    ci = jax.lax.broadcasted_iota(jnp.int32, (_BC, _BC), 1)
    tril = (ci <= ri).astype(jnp.float32)
    cnt_f = cnt.astype(jnp.float32)
    cum_end = jax.lax.dot_general(tril, cnt_f, (((1,), (0,)), ((), ())),
                                  preferred_element_type=jnp.float32)
    cum_end_i = cum_end.astype(jnp.int32)
    cum_start_i = cum_end_i - cnt
    total = jnp.max(cum_end_i)

    p = jax.lax.broadcasted_iota(jnp.int32, (1, _P), 1)
    bc = jnp.sum((p >= cum_end_i).astype(jnp.int32), axis=0, keepdims=True)
    class_iota = jax.lax.broadcasted_iota(jnp.int32, (_BC, _P), 0)
    rowhot = class_iota == bc
    start_p = jnp.sum(jnp.where(rowhot, cum_start_i, 0), axis=0, keepdims=True)
    slot = p - start_p

    keep_hi = jnp.floor(keep * (1.0 / 256.0))
    keep_lo = keep - 256.0 * keep_hi
    rowhot_f = rowhot.astype(jnp.float32)
    dn = (((0,), (0,)), ((), ()))
    cmat_hi = jax.lax.dot_general(keep_hi, rowhot_f, dn,
                                  preferred_element_type=jnp.float32)
    cmat_lo = jax.lax.dot_general(keep_lo, rowhot_f, dn,
                                  preferred_element_type=jnp.float32)
    slot_iota = jax.lax.broadcasted_iota(jnp.int32, (_KPAD, _P), 0)
    slothot = slot_iota == slot
    sel_hi = jnp.sum(jnp.where(slothot, cmat_hi, 0.0), axis=0, keepdims=True)
    sel_lo = jnp.sum(jnp.where(slothot, cmat_lo, 0.0), axis=0, keepdims=True)
    sel = sel_hi.astype(jnp.int32) * 256 + sel_lo.astype(jnp.int32)

    valid = p < total
    b_row = jnp.where(valid, bc >> 3, 0)
    c_row = jnp.where(valid, bc & 7, 0)
    s_row = jnp.where(valid, sel, 0)
    sub = jax.lax.broadcasted_iota(jnp.int32, (8, _P), 0)
    out_ref[...] = jnp.where(sub == 0, b_row,
                             jnp.where(sub == 1, c_row,
                                       jnp.where(sub == 2, s_row, 0)))


def kernel(boxes, scores):
    B, C, N = scores.shape
    sc = scores.reshape(B * C, N)
    sc = jnp.pad(sc, ((0, 0), (0, _NPAD - N)))
    bx = jnp.pad(boxes, ((0, 0), (0, _NPAD - N), (0, 0)))
    x1 = jnp.repeat(bx[:, :, 1], C, axis=0)
    y1 = jnp.repeat(bx[:, :, 0], C, axis=0)
    x2 = jnp.repeat(bx[:, :, 3], C, axis=0)
    y2 = jnp.repeat(bx[:, :, 2], C, axis=0)

    out = pl.pallas_call(
        _nms_kernel,
        out_shape=jax.ShapeDtypeStruct((8, _P), jnp.int32),
    )(x1, y1, x2, y2, sc)

    out_dtype = jnp.asarray(np.zeros((1,), dtype=np.int64)).dtype
    return out[:3].T.astype(out_dtype)
